# Initial kernel scaffold; baseline (speedup 1.0000x reference)
import math

import jax
import jax.numpy as jnp
from jax.experimental import pallas as pl

_B, _C, _T, _H, _W = 16, 768, 8, 12, 30
_N = _H * _W
_K = 120
_NPAD = 384
_N2PAD = 768


def _scores_kernel(x_ref, cls_ref, p_ref):
    xb = x_ref[0]
    cls0 = cls_ref[0]
    s = jnp.dot(cls0, xb)[:, :_N] / math.sqrt(_C)
    p_ref[0] = jax.nn.softmax(s, axis=-1)


def _select_kernel(x_ref, cls_ref, p_all_ref, p_mine_ref, out_ref):
    xb = x_ref[0]
    norm = jnp.max(p_all_ref[...])
    q = p_mine_ref[0] / norm
    qT = jnp.transpose(q)

    row = jax.lax.broadcasted_iota(jnp.int32, (_N, _N), 0)
    col = jax.lax.broadcasted_iota(jnp.int32, (_N, _N), 1)
    cmp = (qT > q) | ((qT == q) & (row < col))
    rank = jnp.sum(cmp.astype(jnp.int32), axis=0, keepdims=True)

    k_iota = jax.lax.broadcasted_iota(jnp.int32, (_K, _N), 0)
    sel = (k_iota == rank).astype(jnp.float32)
    x0 = xb[:, :_N]
    local = jax.lax.dot_general(
        sel, x0, (((1,), (1,)), ((), ())),
        precision=jax.lax.Precision.HIGHEST,
        preferred_element_type=jnp.float32)

    xbT = jnp.transpose(xb)
    glob = xbT[_N:2 * _N, :]

    out_ref[0, 0:1, :] = cls_ref[0, 0:1, :]
    out_ref[0, 1:1 + _K, :] = local
    out_ref[0, 1 + _K:2 + _K, :] = cls_ref[0, 1:2, :]
    out_ref[0, 2 + _K:, :] = glob


def kernel(x, cls_tokens):
    b, c, t, h, w = x.shape
    xr = x.reshape(b, c, t * h * w)

    p = pl.pallas_call(
        _scores_kernel,
        grid=(b,),
        in_specs=[
            pl.BlockSpec((1, c, _NPAD), lambda i: (i, 0, 0)),
            pl.BlockSpec((1, 1, c), lambda i: (i, 0, 0)),
        ],
        out_specs=pl.BlockSpec((1, 1, _N), lambda i: (i, 0, 0)),
        out_shape=jax.ShapeDtypeStruct((b, 1, _N), jnp.float32),
    )(xr, cls_tokens)

    out = pl.pallas_call(
        _select_kernel,
        grid=(b,),
        in_specs=[
            pl.BlockSpec((1, c, _N2PAD), lambda i: (i, 0, 0)),
            pl.BlockSpec((1, 2, c), lambda i: (i, 0, 0)),
            pl.BlockSpec((b, 1, _N), lambda i: (0, 0, 0)),
            pl.BlockSpec((1, 1, _N), lambda i: (i, 0, 0)),
        ],
        out_specs=pl.BlockSpec((1, 2 + _K + _N, c), lambda i: (i, 0, 0)),
        out_shape=jax.ShapeDtypeStruct((b, 2 + _K + _N, c), jnp.float32),
    )(xr, cls_tokens, p, p)
    return out


# speedup vs baseline: 1.0592x; 1.0592x over previous
import math

import jax
import jax.numpy as jnp
from jax.experimental import pallas as pl

_B, _C, _T, _H, _W = 16, 768, 8, 12, 30
_N = _H * _W
_K = 120
_NPAD = 384
_N2PAD = 768


def _scores_kernel(x_ref, cls_ref, p_ref):
    xb = x_ref[0]
    cls0 = cls_ref[0, 0:1, :]
    s = jnp.dot(cls0, xb)[:, :_N] / math.sqrt(_C)
    p_ref[0] = jax.nn.softmax(s, axis=-1)


def _select_kernel(x_ref, cls_ref, p_all_ref, p_mine_ref, out_ref):
    xb = x_ref[0]
    norm = jnp.max(p_all_ref[...])
    q = p_mine_ref[0] / norm
    qT = jnp.transpose(q)

    row = jax.lax.broadcasted_iota(jnp.int32, (_N, _N), 0)
    col = jax.lax.broadcasted_iota(jnp.int32, (_N, _N), 1)
    cmp = (qT > q) | ((qT == q) & (row < col))
    rank = jnp.sum(cmp.astype(jnp.int32), axis=0, keepdims=True)

    k_iota = jax.lax.broadcasted_iota(jnp.int32, (_K, _N), 0)
    sel = (k_iota == rank).astype(jnp.float32)
    x0 = xb[:, :_N]
    local = jax.lax.dot_general(
        sel, x0, (((1,), (1,)), ((), ())),
        precision=jax.lax.Precision.HIGHEST,
        preferred_element_type=jnp.float32)

    xbT = jnp.transpose(xb)
    glob = xbT[_N:2 * _N, :]

    out_ref[0, 0:1, :] = cls_ref[0, 0:1, :]
    out_ref[0, 1:1 + _K, :] = local
    out_ref[0, 1 + _K:2 + _K, :] = cls_ref[0, 1:2, :]
    out_ref[0, 2 + _K:, :] = glob


def kernel(x, cls_tokens):
    b, c, t, h, w = x.shape
    xr = x.reshape(b, c, t * h * w)

    p = pl.pallas_call(
        _scores_kernel,
        grid=(b,),
        in_specs=[
            pl.BlockSpec((1, c, _NPAD), lambda i: (i, 0, 0)),
            pl.BlockSpec((1, t, c), lambda i: (i, 0, 0)),
        ],
        out_specs=pl.BlockSpec((1, 1, _N), lambda i: (i, 0, 0)),
        out_shape=jax.ShapeDtypeStruct((b, 1, _N), jnp.float32),
    )(xr, cls_tokens)

    out = pl.pallas_call(
        _select_kernel,
        grid=(b,),
        in_specs=[
            pl.BlockSpec((1, c, _N2PAD), lambda i: (i, 0, 0)),
            pl.BlockSpec((1, t, c), lambda i: (i, 0, 0)),
            pl.BlockSpec((b, 1, _N), lambda i: (0, 0, 0)),
            pl.BlockSpec((1, 1, _N), lambda i: (i, 0, 0)),
        ],
        out_specs=pl.BlockSpec((1, 2 + _K + _N, c), lambda i: (i, 0, 0)),
        out_shape=jax.ShapeDtypeStruct((b, 2 + _K + _N, c), jnp.float32),
    )(xr, cls_tokens, p, p)
    return out


# speedup vs baseline: 1.5062x; 1.4221x over previous
import math

import jax
import jax.numpy as jnp
from jax.experimental import pallas as pl

_B, _C, _T, _H, _W = 16, 768, 8, 12, 30
_N = _H * _W
_K = 120


def _scores_kernel(x_ref, cls_ref, p_ref):
    x0t = x_ref[0]
    cls0 = cls_ref[0, 0:1, :]
    s = jax.lax.dot_general(
        cls0, x0t, (((1,), (1,)), ((), ())),
        preferred_element_type=jnp.float32) / math.sqrt(_C)
    p_ref[0] = jax.nn.softmax(s, axis=-1)


def _select_kernel(x_ref, cls_ref, p_all_ref, p_mine_ref, out_ref):
    xb = x_ref[0]
    x0t = xb[:, :_C]
    x1t = xb[:, _C:]
    norm = jnp.max(p_all_ref[...])
    q = p_mine_ref[0] / norm
    qT = jnp.transpose(q)

    row = jax.lax.broadcasted_iota(jnp.int32, (_N, _N), 0)
    col = jax.lax.broadcasted_iota(jnp.int32, (_N, _N), 1)
    cmp = (qT > q) | ((qT == q) & (row < col))
    rank = jnp.sum(cmp.astype(jnp.int32), axis=0, keepdims=True)

    k_iota = jax.lax.broadcasted_iota(jnp.int32, (_K, _N), 0)
    sel = (k_iota == rank).astype(jnp.float32)
    local = jax.lax.dot_general(
        sel, x0t, (((1,), (0,)), ((), ())),
        precision=jax.lax.Precision.HIGHEST,
        preferred_element_type=jnp.float32)

    out_ref[0, 0:1, :] = cls_ref[0, 0:1, :]
    out_ref[0, 1:1 + _K, :] = local
    out_ref[0, 1 + _K:2 + _K, :] = cls_ref[0, 1:2, :]
    out_ref[0, 2 + _K:, :] = x1t


def kernel(x, cls_tokens):
    b, c, t, h, w = x.shape
    n = h * w
    xt = jnp.transpose(x, (0, 3, 4, 2, 1))
    x01 = xt[:, :, :, 0:2, :].reshape(b, n, 2 * c)

    p = pl.pallas_call(
        _scores_kernel,
        grid=(b,),
        in_specs=[
            pl.BlockSpec((1, n, c), lambda i: (i, 0, 0)),
            pl.BlockSpec((1, t, c), lambda i: (i, 0, 0)),
        ],
        out_specs=pl.BlockSpec((1, 1, n), lambda i: (i, 0, 0)),
        out_shape=jax.ShapeDtypeStruct((b, 1, n), jnp.float32),
    )(x01, cls_tokens)

    out = pl.pallas_call(
        _select_kernel,
        grid=(b,),
        in_specs=[
            pl.BlockSpec((1, n, 2 * c), lambda i: (i, 0, 0)),
            pl.BlockSpec((1, t, c), lambda i: (i, 0, 0)),
            pl.BlockSpec((b, 1, n), lambda i: (0, 0, 0)),
            pl.BlockSpec((1, 1, n), lambda i: (i, 0, 0)),
        ],
        out_specs=pl.BlockSpec((1, 2 + _K + n, c), lambda i: (i, 0, 0)),
        out_shape=jax.ShapeDtypeStruct((b, 2 + _K + n, c), jnp.float32),
    )(x01, cls_tokens, p, p)
    return out


# speedup vs baseline: 3.5591x; 2.3629x over previous
import math

import jax
import jax.numpy as jnp
from jax.experimental import pallas as pl
from jax.experimental.pallas import tpu as pltpu

_B, _C, _T, _H, _W = 16, 768, 8, 12, 30
_N = _H * _W
_K = 120


def _scores_kernel(x_hbm, cls_ref, p_ref, s0, sem):
    i = pl.program_id(0)
    cp = pltpu.make_async_copy(x_hbm.at[i, :, 0, :], s0, sem)
    cp.start()
    cp.wait()
    x0t = s0[...]
    cls0 = cls_ref[0, 0:1, :]
    s = jax.lax.dot_general(
        cls0, x0t, (((1,), (1,)), ((), ())),
        preferred_element_type=jnp.float32) / math.sqrt(_C)
    p_ref[0] = jax.nn.softmax(s, axis=-1)


def _select_kernel(x_hbm, cls_ref, p_all_ref, p_mine_ref, out_ref, s0, s1,
                   sem0, sem1):
    i = pl.program_id(0)
    cp0 = pltpu.make_async_copy(x_hbm.at[i, :, 0, :], s0, sem0)
    cp1 = pltpu.make_async_copy(x_hbm.at[i, :, 1, :], s1, sem1)
    cp0.start()
    cp1.start()

    norm = jnp.max(p_all_ref[...])
    q = p_mine_ref[0] / norm
    qT = jnp.transpose(q)

    row = jax.lax.broadcasted_iota(jnp.int32, (_N, _N), 0)
    col = jax.lax.broadcasted_iota(jnp.int32, (_N, _N), 1)
    cmp = (qT > q) | ((qT == q) & (row < col))
    rank = jnp.sum(cmp.astype(jnp.int32), axis=0, keepdims=True)

    k_iota = jax.lax.broadcasted_iota(jnp.int32, (_K, _N), 0)
    sel = (k_iota == rank).astype(jnp.float32)

    cp0.wait()
    local = jax.lax.dot_general(
        sel, s0[...], (((1,), (0,)), ((), ())),
        precision=jax.lax.Precision.HIGHEST,
        preferred_element_type=jnp.float32)

    out_ref[0, 0:1, :] = cls_ref[0, 0:1, :]
    out_ref[0, 1:1 + _K, :] = local
    out_ref[0, 1 + _K:2 + _K, :] = cls_ref[0, 1:2, :]
    cp1.wait()
    out_ref[0, 2 + _K:, :] = s1[...]


def kernel(x, cls_tokens):
    b, c, t, h, w = x.shape
    n = h * w
    xt4 = jnp.transpose(x, (0, 3, 4, 2, 1)).reshape(b, n, t, c)

    p = pl.pallas_call(
        _scores_kernel,
        grid=(b,),
        in_specs=[
            pl.BlockSpec(memory_space=pl.ANY),
            pl.BlockSpec((1, t, c), lambda i: (i, 0, 0)),
        ],
        out_specs=pl.BlockSpec((1, 1, n), lambda i: (i, 0, 0)),
        out_shape=jax.ShapeDtypeStruct((b, 1, n), jnp.float32),
        scratch_shapes=[
            pltpu.VMEM((n, c), jnp.float32),
            pltpu.SemaphoreType.DMA,
        ],
    )(xt4, cls_tokens)

    out = pl.pallas_call(
        _select_kernel,
        grid=(b,),
        in_specs=[
            pl.BlockSpec(memory_space=pl.ANY),
            pl.BlockSpec((1, t, c), lambda i: (i, 0, 0)),
            pl.BlockSpec((b, 1, n), lambda i: (0, 0, 0)),
            pl.BlockSpec((1, 1, n), lambda i: (i, 0, 0)),
        ],
        out_specs=pl.BlockSpec((1, 2 + _K + n, c), lambda i: (i, 0, 0)),
        out_shape=jax.ShapeDtypeStruct((b, 2 + _K + n, c), jnp.float32),
        scratch_shapes=[
            pltpu.VMEM((n, c), jnp.float32),
            pltpu.VMEM((n, c), jnp.float32),
            pltpu.SemaphoreType.DMA,
            pltpu.SemaphoreType.DMA,
        ],
    )(xt4, cls_tokens, p, p)
    return out
